# Initial kernel scaffold; baseline (speedup 1.0000x reference)
import jax
import jax.numpy as jnp
from jax.experimental import pallas as pl


def kernel(sequence, state_summary, holder_logits, z_per_step, W1, b1, W2, b2, Wp, bp, Wn, bn, type_embed, time_embed, We, be):
    raise NotImplementedError("write your pallas kernel here")


# speedup vs baseline: 1.8187x; 1.8187x over previous
import functools
import math

import jax
import jax.numpy as jnp
from jax import lax
from jax.experimental import pallas as pl
from jax.experimental.pallas import tpu as pltpu
from jax.experimental.pallas import tpu_sc as plsc

B, T, D = 4, 2048, 1024
NE = 32
NT = 7
ME = 32
THRESH = 0.4
ZBW = 0.15
MAX_TIME = 512

TT = 256
NROWS = B * T
NSEL = B * ME

_NEG_INF = float("-inf")


def _heads_body(x_ref, w1_ref, b1_ref, w2_ref, b2_ref, wp_ref, bp_ref,
                wn_ref, bn_ref, etl_ref, prev_ref, next_ref):
    x = x_ref[...]
    h = jnp.dot(x, w1_ref[...], preferred_element_type=jnp.float32) + b1_ref[...]
    g = 0.5 * h * (lax.erf(h / math.sqrt(2.0)) + 1.0)
    etl_ref[...] = jnp.dot(g, w2_ref[...],
                           preferred_element_type=jnp.float32) + b2_ref[...]
    prev_ref[...] = jnp.dot(x, wp_ref[...],
                            preferred_element_type=jnp.float32) + bp_ref[...]
    next_ref[...] = jnp.dot(x, wn_ref[...],
                            preferred_element_type=jnp.float32) + bn_ref[...]


def _heads(seq2d, W1, b1, W2, b2, Wp, bp, Wn, bn):
    nt = NROWS // TT
    row_spec = lambda w: pl.BlockSpec((TT, w), lambda t: (t, 0))
    full = lambda a: pl.BlockSpec(a.shape, lambda t: (0,) * a.ndim)
    return pl.pallas_call(
        _heads_body,
        grid=(nt,),
        in_specs=[row_spec(D)] + [full(a) for a in (W1, b1, W2, b2, Wp, bp, Wn, bn)],
        out_specs=[row_spec(NT), row_spec(NE), row_spec(NE)],
        out_shape=[
            jax.ShapeDtypeStruct((NROWS, NT), jnp.float32),
            jax.ShapeDtypeStruct((NROWS, NE), jnp.float32),
            jax.ShapeDtypeStruct((NROWS, NE), jnp.float32),
        ],
    )(seq2d, W1, b1, W2, b2, Wp, bp, Wn, bn)


def _select_body(etl_ref, z_ref, scores_ref, mask_ref, times_ref, tids_ref,
                 tysel_ref, idxflat_ref, idxtime_ref):
    etl = etl_ref[...]
    z = z_ref[...]

    emax = jnp.max(etl, axis=2, keepdims=True)
    ee = jnp.exp(etl - emax)
    esum = jnp.sum(ee, axis=2)
    non_none = 1.0 - ee[:, :, 0] / esum

    it3 = lax.broadcasted_iota(jnp.int32, (B, T, NT), 2)
    ty = jnp.min(jnp.where(etl == emax, it3, NT), axis=2)

    zb = jnp.max(jnp.abs(z), axis=2)
    zmax = jnp.max(zb, axis=1, keepdims=True)
    scores = non_none + ZBW * zb / jnp.maximum(zmax, 1.0)
    scores_ref[...] = scores

    iota_t = lax.broadcasted_iota(jnp.int32, (B, T), 1)

    key0 = jnp.where(scores >= THRESH, scores, _NEG_INF)

    def ext_step(_, carry):
        key, selmask, cnt = carry
        m = jnp.max(key, axis=1, keepdims=True)
        has = m > _NEG_INF
        hit = key == m
        idx = jnp.min(jnp.where(hit, iota_t, T), axis=1, keepdims=True)
        pick = (iota_t == idx) & has
        return (jnp.where(pick, _NEG_INF, key), selmask | pick.astype(jnp.int32),
                cnt + has.astype(jnp.int32))

    selmask0 = jnp.zeros((B, T), jnp.int32)
    cnt0 = jnp.zeros((B, 1), jnp.int32)
    _, selmask, n_eff = lax.fori_loop(0, ME, ext_step, (key0, selmask0, cnt0))

    ikey = jnp.where(selmask > 0, iota_t, T)
    ch_cols, ty_cols = [], []
    for _ in range(ME):
        idx = jnp.min(ikey, axis=1, keepdims=True)
        ch_cols.append(idx)
        ty_cols.append(jnp.max(jnp.where(iota_t == idx, ty, 0),
                               axis=1, keepdims=True))
        ikey = jnp.where(iota_t == idx, T, ikey)
    ch = jnp.concatenate(ch_cols, axis=1)
    tysel = jnp.concatenate(ty_cols, axis=1)

    empty = n_eff == 0
    slot = lax.broadcasted_iota(jnp.int32, (B, ME), 1)
    ch = jnp.where(empty, slot, ch)
    tysel = jnp.where(empty, ty[:, :ME], tysel)
    n = jnp.where(empty, ME, n_eff)
    mask = slot < n

    mask_ref[...] = mask.astype(jnp.int32)
    times_ref[...] = jnp.where(mask, ch, 0)
    tids_ref[...] = jnp.where(mask, tysel, 0)
    tysel_ref[...] = tysel
    bi = lax.broadcasted_iota(jnp.int32, (B, ME), 0)
    idxflat_ref[...] = bi * T + jnp.minimum(ch, T - 1)
    idxtime_ref[...] = jnp.clip(ch, 0, MAX_TIME - 1)


def _select(etl3, z):
    i32 = jnp.int32
    return pl.pallas_call(
        _select_body,
        out_shape=[
            jax.ShapeDtypeStruct((B, T), jnp.float32),
            jax.ShapeDtypeStruct((B, ME), i32),
            jax.ShapeDtypeStruct((B, ME), i32),
            jax.ShapeDtypeStruct((B, ME), i32),
            jax.ShapeDtypeStruct((B, ME), i32),
            jax.ShapeDtypeStruct((B, ME), i32),
            jax.ShapeDtypeStruct((B, ME), i32),
        ],
    )(etl3, z)


_NC = 2
_NWORK = 16
_RPW = NSEL // _NWORK


def _sc_gather_body(seq_hbm, ss_hbm, hl_hbm, te_hbm, idxf_hbm, idxt_hbm,
                    seqo_hbm, sso_hbm, hlo_hbm, teo_hbm,
                    idxf_v, idxt_v, rows_v, rows2_v, hlrows_v, terows_v, sem):
    wid = lax.axis_index("s") * _NC + lax.axis_index("c")

    @pl.when(wid < _NWORK)
    def _():
        base = wid * _RPW
        pltpu.sync_copy(idxf_hbm.at[pl.ds(base, _RPW)], idxf_v)
        pltpu.sync_copy(idxt_hbm.at[pl.ds(base, _RPW)], idxt_v)
        pltpu.async_copy(seq_hbm.at[idxf_v], rows_v, sem).wait()
        pltpu.sync_copy(rows_v, seqo_hbm.at[pl.ds(base, _RPW)])
        pltpu.async_copy(ss_hbm.at[idxf_v], rows2_v, sem).wait()
        pltpu.sync_copy(rows2_v, sso_hbm.at[pl.ds(base, _RPW)])
        pltpu.async_copy(hl_hbm.at[idxf_v], hlrows_v, sem).wait()
        pltpu.sync_copy(hlrows_v, hlo_hbm.at[pl.ds(base, _RPW)])
        pltpu.async_copy(te_hbm.at[idxt_v], terows_v, sem).wait()
        pltpu.sync_copy(terows_v, teo_hbm.at[pl.ds(base, _RPW)])


def _sc_gather(seq2d, ss2d, hl2d, time_embed, idx_flat, idx_time):
    f32 = jnp.float32
    mesh = plsc.VectorSubcoreMesh(core_axis_name="c", subcore_axis_name="s")
    k = pl.kernel(
        _sc_gather_body,
        out_type=[
            jax.ShapeDtypeStruct((NSEL, D), f32),
            jax.ShapeDtypeStruct((NSEL, D), f32),
            jax.ShapeDtypeStruct((NSEL, 128), f32),
            jax.ShapeDtypeStruct((NSEL, D), f32),
        ],
        mesh=mesh,
        scratch_types=[
            pltpu.VMEM((_RPW,), jnp.int32),
            pltpu.VMEM((_RPW,), jnp.int32),
            pltpu.VMEM((_RPW, D), f32),
            pltpu.VMEM((_RPW, D), f32),
            pltpu.VMEM((_RPW, 128), f32),
            pltpu.VMEM((_RPW, D), f32),
            pltpu.SemaphoreType.DMA,
        ],
    )
    return k(seq2d, ss2d, hl2d, time_embed, idx_flat, idx_time)


def _entries_body(seqs_ref, sss_ref, hls_ref, tes_ref, tys_ref, msk_ref,
                  we1_ref, we2_ref, we3_ref, be_ref, temb_ref, out_ref):
    hl = hls_ref[:, :NE]
    hmax = jnp.max(hl, axis=1, keepdims=True)
    he = jnp.exp(hl - hmax)
    hp = he / jnp.sum(he, axis=1, keepdims=True)

    acc = jnp.dot(seqs_ref[...], we1_ref[...], preferred_element_type=jnp.float32)
    acc = acc + jnp.dot(sss_ref[...], we2_ref[...],
                        preferred_element_type=jnp.float32)
    acc = acc + jnp.dot(hp, we3_ref[...], preferred_element_type=jnp.float32)
    acc = acc + be_ref[...]

    onehot = (lax.broadcasted_iota(jnp.int32, (NSEL, NT), 1)
              == tys_ref[...]).astype(jnp.float32)
    acc = acc + jnp.dot(onehot, temb_ref[...], preferred_element_type=jnp.float32)
    acc = acc + tes_ref[...]
    out_ref[...] = jnp.where(msk_ref[...] > 0, acc, 0.0)


def _entries(seq_sel, ss_sel, hl_sel, te_sel, tysel, mask_i, We, be, type_embed):
    we1 = We[:D]
    we2 = We[D:2 * D]
    we3 = We[2 * D:]
    return pl.pallas_call(
        _entries_body,
        out_shape=jax.ShapeDtypeStruct((NSEL, D), jnp.float32),
    )(seq_sel, ss_sel, hl_sel, te_sel,
      tysel.reshape(NSEL, 1), mask_i.reshape(NSEL, 1),
      we1, we2, we3, be.reshape(1, D), type_embed)


@functools.partial(jax.jit, static_argnums=())
def kernel(sequence, state_summary, holder_logits, z_per_step, W1, b1, W2, b2,
           Wp, bp, Wn, bn, type_embed, time_embed, We, be):
    seq2d = sequence.reshape(NROWS, D)
    etl2d, prev2d, next2d = _heads(
        seq2d, W1, b1.reshape(1, D), W2, b2.reshape(1, NT),
        Wp, bp.reshape(1, NE), Wn, bn.reshape(1, NE))

    etl = etl2d.reshape(B, T, NT)
    (scores, mask_i, times, type_ids, tysel, idx_flat, idx_time) = _select(
        etl, z_per_step)

    hl_padded = jnp.pad(holder_logits.reshape(NROWS, NE),
                        ((0, 0), (0, 128 - NE)))
    seq_sel, ss_sel, hl_sel, te_sel = _sc_gather(
        seq2d, state_summary.reshape(NROWS, D),
        hl_padded, time_embed,
        idx_flat.reshape(NSEL), idx_time.reshape(NSEL))

    ent2d = _entries(seq_sel, ss_sel, hl_sel, te_sel, tysel, mask_i,
                     We, be, type_embed)

    return (ent2d.reshape(B, ME, D), mask_i.astype(jnp.bool_), times,
            type_ids, etl, prev2d.reshape(B, T, NE), next2d.reshape(B, T, NE),
            scores)


# speedup vs baseline: 2.0096x; 1.1050x over previous
import functools
import math

import jax
import jax.numpy as jnp
from jax import lax
from jax.experimental import pallas as pl
from jax.experimental.pallas import tpu as pltpu
from jax.experimental.pallas import tpu_sc as plsc

B, T, D = 4, 2048, 1024
NE = 32
NT = 7
ME = 32
THRESH = 0.4
ZBW = 0.15
MAX_TIME = 512

TT = 512
NROWS = B * T
NSEL = B * ME

_NEG_INF = float("-inf")


def _heads_body(x_ref, z_ref, hl_ref, w1_ref, b1_ref, w2_ref, b2_ref,
                wp_ref, bp_ref, wn_ref, bn_ref,
                etl_ref, prev_ref, next_ref, nn_ref, zb_ref, ty_ref, hlp_ref):
    x = x_ref[...]
    xb = x.astype(jnp.bfloat16)
    h = jnp.dot(xb, w1_ref[...], preferred_element_type=jnp.float32) + b1_ref[...]
    g = 0.5 * h * (lax.erf(h / math.sqrt(2.0)) + 1.0)
    etl = jnp.dot(g.astype(jnp.bfloat16), w2_ref[...],
                  preferred_element_type=jnp.float32) + b2_ref[...]
    etl_ref[...] = etl
    prev_ref[...] = jnp.dot(xb, wp_ref[...],
                            preferred_element_type=jnp.float32) + bp_ref[...]
    next_ref[...] = jnp.dot(xb, wn_ref[...],
                            preferred_element_type=jnp.float32) + bn_ref[...]

    emax = jnp.max(etl, axis=1, keepdims=True)
    ee = jnp.exp(etl - emax)
    esum = jnp.sum(ee, axis=1, keepdims=True)
    nn = 1.0 - ee[:, :1] / esum
    it2 = lax.broadcasted_iota(jnp.int32, (TT, NT), 1)
    ty = jnp.min(jnp.where(etl == emax, it2, NT), axis=1, keepdims=True)
    zb = jnp.max(jnp.abs(z_ref[...]), axis=1, keepdims=True)

    nn_ref[...] = nn.T
    zb_ref[...] = zb.T
    ty_ref[...] = ty.T

    hl = hl_ref[...]
    hlp_ref[...] = jnp.concatenate(
        [hl, jnp.zeros((TT, 128 - NE), jnp.float32)], axis=1)


def _heads(seq2d, z2d, hl2d, W1, b1, W2, b2, Wp, bp, Wn, bn):
    nt = NROWS // TT
    row = lambda w: pl.BlockSpec((TT, w), lambda t: (t, 0))
    vec = lambda: pl.BlockSpec((1, TT), lambda t: (0, t))
    full = lambda a: pl.BlockSpec(a.shape, lambda t: (0,) * a.ndim)
    f32 = jnp.float32
    return pl.pallas_call(
        _heads_body,
        grid=(nt,),
        in_specs=[row(D), row(NE), row(NE)]
        + [full(a) for a in (W1, b1, W2, b2, Wp, bp, Wn, bn)],
        out_specs=[row(NT), row(NE), row(NE), vec(), vec(), vec(), row(128)],
        out_shape=[
            jax.ShapeDtypeStruct((NROWS, NT), f32),
            jax.ShapeDtypeStruct((NROWS, NE), f32),
            jax.ShapeDtypeStruct((NROWS, NE), f32),
            jax.ShapeDtypeStruct((1, NROWS), f32),
            jax.ShapeDtypeStruct((1, NROWS), f32),
            jax.ShapeDtypeStruct((1, NROWS), jnp.int32),
            jax.ShapeDtypeStruct((NROWS, 128), f32),
        ],
    )(seq2d, z2d, hl2d, W1, b1, W2, b2, Wp, bp, Wn, bn)


def _select_body(nn_ref, zbr_ref, ty_ref, scores_ref, mask_ref, times_ref,
                 tids_ref, tysel_ref, idxflat_ref, idxtime_ref):
    nn = nn_ref[...]
    zb = zbr_ref[...]
    ty = ty_ref[...]

    zmax = jnp.max(zb, axis=1, keepdims=True)
    scores = nn + ZBW * zb / jnp.maximum(zmax, 1.0)
    scores_ref[...] = scores

    iota_t = lax.broadcasted_iota(jnp.int32, (B, T), 1)

    key0 = jnp.where(scores >= THRESH, scores, _NEG_INF)

    def ext_step(_, carry):
        key, selmask, cnt = carry
        m = jnp.max(key, axis=1, keepdims=True)
        has = m > _NEG_INF
        hit = key == m
        idx = jnp.min(jnp.where(hit, iota_t, T), axis=1, keepdims=True)
        pick = (iota_t == idx) & has
        return (jnp.where(pick, _NEG_INF, key), selmask | pick.astype(jnp.int32),
                cnt + has.astype(jnp.int32))

    selmask0 = jnp.zeros((B, T), jnp.int32)
    cnt0 = jnp.zeros((B, 1), jnp.int32)
    _, selmask, n_eff = lax.fori_loop(0, ME, ext_step, (key0, selmask0, cnt0))

    ikey = jnp.where(selmask > 0, iota_t, T)
    ch_cols, ty_cols = [], []
    for _ in range(ME):
        idx = jnp.min(ikey, axis=1, keepdims=True)
        ch_cols.append(idx)
        ty_cols.append(jnp.max(jnp.where(iota_t == idx, ty, 0),
                               axis=1, keepdims=True))
        ikey = jnp.where(iota_t == idx, T, ikey)
    ch = jnp.concatenate(ch_cols, axis=1)
    tysel = jnp.concatenate(ty_cols, axis=1)

    empty = n_eff == 0
    slot = lax.broadcasted_iota(jnp.int32, (B, ME), 1)
    ch = jnp.where(empty, slot, ch)
    tysel = jnp.where(empty, ty[:, :ME], tysel)
    n = jnp.where(empty, ME, n_eff)
    mask = slot < n

    mask_ref[...] = mask.astype(jnp.int32)
    times_ref[...] = jnp.where(mask, ch, 0)
    tids_ref[...] = jnp.where(mask, tysel, 0)
    tysel_ref[...] = tysel
    bi = lax.broadcasted_iota(jnp.int32, (B, ME), 0)
    idxflat_ref[...] = bi * T + jnp.minimum(ch, T - 1)
    idxtime_ref[...] = jnp.clip(ch, 0, MAX_TIME - 1)


def _select(nn, zb, ty):
    i32 = jnp.int32
    return pl.pallas_call(
        _select_body,
        out_shape=[
            jax.ShapeDtypeStruct((B, T), jnp.float32),
            jax.ShapeDtypeStruct((B, ME), i32),
            jax.ShapeDtypeStruct((B, ME), i32),
            jax.ShapeDtypeStruct((B, ME), i32),
            jax.ShapeDtypeStruct((B, ME), i32),
            jax.ShapeDtypeStruct((B, ME), i32),
            jax.ShapeDtypeStruct((B, ME), i32),
        ],
    )(nn, zb, ty)


_NC = 2
_NWORK = 16
_RPW = NSEL // _NWORK


def _sc_gather_body(seq_hbm, ss_hbm, hl_hbm, te_hbm, idxf_hbm, idxt_hbm,
                    seqo_hbm, sso_hbm, hlo_hbm, teo_hbm,
                    idxf_v, idxt_v, buf1_v, buf2_v, hlbuf_v, sem):
    wid = lax.axis_index("s") * _NC + lax.axis_index("c")
    j = lax.rem(wid, _NWORK)
    base = j * _RPW

    @pl.when(wid < _NWORK)
    def _():
        pltpu.sync_copy(idxf_hbm.at[pl.ds(base, _RPW)], idxf_v)
        c1 = pltpu.async_copy(seq_hbm.at[idxf_v], buf1_v, sem)
        c2 = pltpu.async_copy(hl_hbm.at[idxf_v], hlbuf_v, sem)
        c1.wait()
        c2.wait()
        o1 = pltpu.async_copy(buf1_v, seqo_hbm.at[pl.ds(base, _RPW)], sem)
        o2 = pltpu.async_copy(hlbuf_v, hlo_hbm.at[pl.ds(base, _RPW)], sem)
        o1.wait()
        o2.wait()

    @pl.when(wid >= _NWORK)
    def _():
        pltpu.sync_copy(idxf_hbm.at[pl.ds(base, _RPW)], idxf_v)
        pltpu.sync_copy(idxt_hbm.at[pl.ds(base, _RPW)], idxt_v)
        c1 = pltpu.async_copy(ss_hbm.at[idxf_v], buf1_v, sem)
        c2 = pltpu.async_copy(te_hbm.at[idxt_v], buf2_v, sem)
        c1.wait()
        c2.wait()
        o1 = pltpu.async_copy(buf1_v, sso_hbm.at[pl.ds(base, _RPW)], sem)
        o2 = pltpu.async_copy(buf2_v, teo_hbm.at[pl.ds(base, _RPW)], sem)
        o1.wait()
        o2.wait()


def _sc_gather(seq2d, ss2d, hlp, time_embed, idx_flat, idx_time):
    f32 = jnp.float32
    mesh = plsc.VectorSubcoreMesh(core_axis_name="c", subcore_axis_name="s")
    k = pl.kernel(
        _sc_gather_body,
        out_type=[
            jax.ShapeDtypeStruct((NSEL, D), f32),
            jax.ShapeDtypeStruct((NSEL, D), f32),
            jax.ShapeDtypeStruct((NSEL, 128), f32),
            jax.ShapeDtypeStruct((NSEL, D), f32),
        ],
        mesh=mesh,
        scratch_types=[
            pltpu.VMEM((_RPW,), jnp.int32),
            pltpu.VMEM((_RPW,), jnp.int32),
            pltpu.VMEM((_RPW, D), f32),
            pltpu.VMEM((_RPW, D), f32),
            pltpu.VMEM((_RPW, 128), f32),
            pltpu.SemaphoreType.DMA,
        ],
    )
    return k(seq2d, ss2d, hlp, time_embed, idx_flat, idx_time)


def _entries_body(seqs_ref, sss_ref, hls_ref, tes_ref, tys_ref, msk_ref,
                  we1_ref, we2_ref, we3_ref, be_ref, temb_ref, out_ref):
    hl = hls_ref[:, :NE]
    hmax = jnp.max(hl, axis=1, keepdims=True)
    he = jnp.exp(hl - hmax)
    hp = he / jnp.sum(he, axis=1, keepdims=True)

    acc = jnp.dot(seqs_ref[...].astype(jnp.bfloat16), we1_ref[...],
                  preferred_element_type=jnp.float32)
    acc = acc + jnp.dot(sss_ref[...].astype(jnp.bfloat16), we2_ref[...],
                        preferred_element_type=jnp.float32)
    acc = acc + jnp.dot(hp.astype(jnp.bfloat16), we3_ref[...],
                        preferred_element_type=jnp.float32)
    acc = acc + be_ref[...]

    onehot = (lax.broadcasted_iota(jnp.int32, (NSEL, NT), 1)
              == tys_ref[...]).astype(jnp.float32)
    acc = acc + jnp.dot(onehot.astype(jnp.bfloat16), temb_ref[...],
                        preferred_element_type=jnp.float32)
    acc = acc + tes_ref[...]
    out_ref[...] = jnp.where(msk_ref[...] > 0, acc, 0.0)


def _entries(seq_sel, ss_sel, hl_sel, te_sel, tysel, mask_i, we1, we2, we3,
             be, type_embed):
    return pl.pallas_call(
        _entries_body,
        out_shape=jax.ShapeDtypeStruct((NSEL, D), jnp.float32),
    )(seq_sel, ss_sel, hl_sel, te_sel,
      tysel.reshape(NSEL, 1), mask_i.reshape(NSEL, 1),
      we1, we2, we3, be.reshape(1, D), type_embed)


@functools.partial(jax.jit, static_argnums=())
def kernel(sequence, state_summary, holder_logits, z_per_step, W1, b1, W2, b2,
           Wp, bp, Wn, bn, type_embed, time_embed, We, be):
    bf16 = jnp.bfloat16
    seq2d = sequence.reshape(NROWS, D)
    etl2d, prev2d, next2d, nn_row, zb_row, ty_row, hlp = _heads(
        seq2d, z_per_step.reshape(NROWS, NE), holder_logits.reshape(NROWS, NE),
        W1.astype(bf16), b1.reshape(1, D), W2.astype(bf16), b2.reshape(1, NT),
        Wp.astype(bf16), bp.reshape(1, NE), Wn.astype(bf16), bn.reshape(1, NE))

    (scores, mask_i, times, type_ids, tysel, idx_flat, idx_time) = _select(
        nn_row.reshape(B, T), zb_row.reshape(B, T), ty_row.reshape(B, T))

    seq_sel, ss_sel, hl_sel, te_sel = _sc_gather(
        seq2d, state_summary.reshape(NROWS, D), hlp, time_embed,
        idx_flat.reshape(NSEL), idx_time.reshape(NSEL))

    ent2d = _entries(seq_sel, ss_sel, hl_sel, te_sel, tysel, mask_i,
                     We[:D].astype(bf16), We[D:2 * D].astype(bf16),
                     We[2 * D:].astype(bf16), be, type_embed.astype(bf16))

    return (ent2d.reshape(B, ME, D), mask_i.astype(jnp.bool_), times,
            type_ids, etl2d.reshape(B, T, NT), prev2d.reshape(B, T, NE),
            next2d.reshape(B, T, NE), scores)


# speedup vs baseline: 2.2704x; 1.1298x over previous
import functools
import math

import jax
import jax.numpy as jnp
from jax import lax
from jax.experimental import pallas as pl
from jax.experimental.pallas import tpu as pltpu
from jax.experimental.pallas import tpu_sc as plsc

B, T, D = 4, 2048, 1024
NE = 32
NT = 7
ME = 32
THRESH = 0.4
ZBW = 0.15
MAX_TIME = 512

TT = 512
NROWS = B * T
NSEL = B * ME
TPB = T // TT

_NEG_INF = float("-inf")


def _heads_body(x_ref, z_ref, hl_ref, w1_ref, b1_ref, w2_ref, b2_ref,
                wp_ref, bp_ref, wn_ref, bn_ref,
                etl_ref, prev_ref, next_ref, nn_ref, zb_ref, ty_ref, hlp_ref,
                w1b_ref):
    @pl.when(pl.program_id(0) == 0)
    def _():
        w1b_ref[...] = w1_ref[...].astype(jnp.bfloat16)

    x = x_ref[...]
    xb = x.astype(jnp.bfloat16)
    h = jnp.dot(xb, w1b_ref[...], preferred_element_type=jnp.float32) + b1_ref[...]
    g = 0.5 * h * (lax.erf(h / math.sqrt(2.0)) + 1.0)
    etl = jnp.dot(g.astype(jnp.bfloat16), w2_ref[...].astype(jnp.bfloat16),
                  preferred_element_type=jnp.float32) + b2_ref[...]
    etl_ref[...] = etl
    prev_ref[...] = jnp.dot(xb, wp_ref[...].astype(jnp.bfloat16),
                            preferred_element_type=jnp.float32) + bp_ref[...]
    next_ref[...] = jnp.dot(xb, wn_ref[...].astype(jnp.bfloat16),
                            preferred_element_type=jnp.float32) + bn_ref[...]

    emax = jnp.max(etl, axis=1, keepdims=True)
    ee = jnp.exp(etl - emax)
    esum = jnp.sum(ee, axis=1, keepdims=True)
    nn = 1.0 - ee[:, :1] / esum
    it2 = lax.broadcasted_iota(jnp.int32, (TT, NT), 1)
    ty = jnp.min(jnp.where(etl == emax, it2, NT), axis=1, keepdims=True)
    zb = jnp.max(jnp.abs(z_ref[...]), axis=1, keepdims=True)

    nn_ref[...] = nn.T.reshape(1, 1, TT)
    zb_ref[...] = zb.T.reshape(1, 1, TT)
    ty_ref[...] = ty.T.reshape(1, 1, TT)

    hl = hl_ref[...]
    hlp_ref[...] = jnp.concatenate(
        [hl, jnp.zeros((TT, 128 - NE), jnp.float32)], axis=1)


def _heads(seq2d, z2d, hl2d, W1, b1, W2, b2, Wp, bp, Wn, bn):
    nt = NROWS // TT
    row = lambda w: pl.BlockSpec((TT, w), lambda t: (t, 0))
    vec = lambda: pl.BlockSpec((1, 1, TT), lambda t: (t // TPB, 0, t % TPB))
    full = lambda a: pl.BlockSpec(a.shape, lambda t: (0,) * a.ndim)
    f32 = jnp.float32
    return pl.pallas_call(
        _heads_body,
        grid=(nt,),
        in_specs=[row(D), row(NE), row(NE)]
        + [full(a) for a in (W1, b1, W2, b2, Wp, bp, Wn, bn)],
        out_specs=[row(NT), row(NE), row(NE), vec(), vec(), vec(), row(128)],
        out_shape=[
            jax.ShapeDtypeStruct((NROWS, NT), f32),
            jax.ShapeDtypeStruct((NROWS, NE), f32),
            jax.ShapeDtypeStruct((NROWS, NE), f32),
            jax.ShapeDtypeStruct((B, 1, T), f32),
            jax.ShapeDtypeStruct((B, 1, T), f32),
            jax.ShapeDtypeStruct((B, 1, T), jnp.int32),
            jax.ShapeDtypeStruct((NROWS, 128), f32),
        ],
        scratch_shapes=[pltpu.VMEM((D, D), jnp.bfloat16)],
    )(seq2d, z2d, hl2d, W1, b1, W2, b2, Wp, bp, Wn, bn)


def _select_body(nn_ref, zbr_ref, ty_ref, scores_ref, mask_ref, times_ref,
                 tids_ref, idxflat_ref, idxtime_ref, idxty_ref, idxmsk_ref):
    nn = nn_ref[...].reshape(B, T)
    zb = zbr_ref[...].reshape(B, T)
    ty = ty_ref[...].reshape(B, T)

    zmax = jnp.max(zb, axis=1, keepdims=True)
    scores = nn + ZBW * zb / jnp.maximum(zmax, 1.0)
    scores_ref[...] = scores

    iota_t = lax.broadcasted_iota(jnp.int32, (B, T), 1)

    key0 = jnp.where(scores >= THRESH, scores, _NEG_INF)

    def ext_step(_, carry):
        key, selmask, cnt = carry
        m = jnp.max(key, axis=1, keepdims=True)
        has = m > _NEG_INF
        hit = key == m
        idx = jnp.min(jnp.where(hit, iota_t, T), axis=1, keepdims=True)
        pick = (iota_t == idx) & has
        return (jnp.where(pick, _NEG_INF, key), selmask | pick.astype(jnp.int32),
                cnt + has.astype(jnp.int32))

    selmask0 = jnp.zeros((B, T), jnp.int32)
    cnt0 = jnp.zeros((B, 1), jnp.int32)
    _, selmask, n_eff = lax.fori_loop(0, ME, ext_step, (key0, selmask0, cnt0))

    ikey = jnp.where(selmask > 0, iota_t, T)
    ch_cols, ty_cols = [], []
    for _ in range(ME):
        idx = jnp.min(ikey, axis=1, keepdims=True)
        ch_cols.append(idx)
        ty_cols.append(jnp.max(jnp.where(iota_t == idx, ty, 0),
                               axis=1, keepdims=True))
        ikey = jnp.where(iota_t == idx, T, ikey)
    ch = jnp.concatenate(ch_cols, axis=1)
    tysel = jnp.concatenate(ty_cols, axis=1)

    empty = n_eff == 0
    slot = lax.broadcasted_iota(jnp.int32, (B, ME), 1)
    ch = jnp.where(empty, slot, ch)
    tysel = jnp.where(empty, ty[:, :ME], tysel)
    n = jnp.where(empty, ME, n_eff)
    mask = slot < n
    mask_i = mask.astype(jnp.int32)

    mask_ref[...] = mask_i
    times_ref[...] = jnp.where(mask, ch, 0)
    tids_ref[...] = jnp.where(mask, tysel, 0)

    def flat_row(a):
        return jnp.concatenate([a[b:b + 1, :] for b in range(B)], axis=1)

    bi = lax.broadcasted_iota(jnp.int32, (B, ME), 0)
    idxflat_ref[...] = flat_row(bi * T + jnp.minimum(ch, T - 1))
    idxtime_ref[...] = flat_row(jnp.clip(ch, 0, MAX_TIME - 1))
    idxty_ref[...] = flat_row(tysel)
    idxmsk_ref[...] = flat_row(mask_i)


def _select(nn, zb, ty):
    i32 = jnp.int32
    return pl.pallas_call(
        _select_body,
        out_shape=[
            jax.ShapeDtypeStruct((B, T), jnp.float32),
            jax.ShapeDtypeStruct((B, ME), i32),
            jax.ShapeDtypeStruct((B, ME), i32),
            jax.ShapeDtypeStruct((B, ME), i32),
            jax.ShapeDtypeStruct((1, NSEL), i32),
            jax.ShapeDtypeStruct((1, NSEL), i32),
            jax.ShapeDtypeStruct((1, NSEL), i32),
            jax.ShapeDtypeStruct((1, NSEL), i32),
        ],
    )(nn, zb, ty)


_NC = 2
_NWORK = 16
_RPW = NSEL // _NWORK


def _sc_gather_body(seq_hbm, ss_hbm, hl_hbm, te_hbm, tye_hbm, mrow_hbm,
                    idxf_hbm, idxt_hbm, idxy_hbm, idxm_hbm,
                    seqo_hbm, sso_hbm, hlo_hbm, teo_hbm, tyeo_hbm, mrowo_hbm,
                    ia_v, ib_v, ic_v, buf1_v, buf2_v, sbuf_v, sem):
    wid = lax.axis_index("s") * _NC + lax.axis_index("c")
    j = lax.rem(wid, _NWORK)
    base = j * _RPW

    @pl.when(wid < _NWORK)
    def _():
        pltpu.sync_copy(idxf_hbm.at[0, pl.ds(base, _RPW)], ia_v)
        pltpu.sync_copy(idxy_hbm.at[0, pl.ds(base, _RPW)], ic_v)
        c1 = pltpu.async_copy(seq_hbm.at[ia_v], buf1_v, sem)
        c2 = pltpu.async_copy(hl_hbm.at[ia_v], sbuf_v, sem)
        c3 = pltpu.async_copy(tye_hbm.at[ic_v], buf2_v, sem)
        c1.wait()
        c2.wait()
        c3.wait()
        o1 = pltpu.async_copy(buf1_v, seqo_hbm.at[pl.ds(base, _RPW)], sem)
        o2 = pltpu.async_copy(sbuf_v, hlo_hbm.at[pl.ds(base, _RPW)], sem)
        o3 = pltpu.async_copy(buf2_v, tyeo_hbm.at[pl.ds(base, _RPW)], sem)
        o1.wait()
        o2.wait()
        o3.wait()

    @pl.when(wid >= _NWORK)
    def _():
        pltpu.sync_copy(idxf_hbm.at[0, pl.ds(base, _RPW)], ia_v)
        pltpu.sync_copy(idxt_hbm.at[0, pl.ds(base, _RPW)], ib_v)
        pltpu.sync_copy(idxm_hbm.at[0, pl.ds(base, _RPW)], ic_v)
        c1 = pltpu.async_copy(ss_hbm.at[ia_v], buf1_v, sem)
        c2 = pltpu.async_copy(te_hbm.at[ib_v], buf2_v, sem)
        c3 = pltpu.async_copy(mrow_hbm.at[ic_v], sbuf_v, sem)
        c1.wait()
        c2.wait()
        c3.wait()
        o1 = pltpu.async_copy(buf1_v, sso_hbm.at[pl.ds(base, _RPW)], sem)
        o2 = pltpu.async_copy(buf2_v, teo_hbm.at[pl.ds(base, _RPW)], sem)
        o3 = pltpu.async_copy(sbuf_v, mrowo_hbm.at[pl.ds(base, _RPW)], sem)
        o1.wait()
        o2.wait()
        o3.wait()


def _sc_gather(seq2d, ss2d, hlp, time_embed, type_embed, mrow_tbl,
               idx_flat, idx_time, idx_ty, idx_msk):
    f32 = jnp.float32
    mesh = plsc.VectorSubcoreMesh(core_axis_name="c", subcore_axis_name="s")
    k = pl.kernel(
        _sc_gather_body,
        out_type=[
            jax.ShapeDtypeStruct((NSEL, D), f32),
            jax.ShapeDtypeStruct((NSEL, D), f32),
            jax.ShapeDtypeStruct((NSEL, 128), f32),
            jax.ShapeDtypeStruct((NSEL, D), f32),
            jax.ShapeDtypeStruct((NSEL, D), f32),
            jax.ShapeDtypeStruct((NSEL, 128), f32),
        ],
        mesh=mesh,
        scratch_types=[
            pltpu.VMEM((_RPW,), jnp.int32),
            pltpu.VMEM((_RPW,), jnp.int32),
            pltpu.VMEM((_RPW,), jnp.int32),
            pltpu.VMEM((_RPW, D), f32),
            pltpu.VMEM((_RPW, D), f32),
            pltpu.VMEM((_RPW, 128), f32),
            pltpu.SemaphoreType.DMA,
        ],
    )
    return k(seq2d, ss2d, hlp, time_embed, type_embed, mrow_tbl,
             idx_flat, idx_time, idx_ty, idx_msk)


def _entries_body(seqs_ref, sss_ref, hls_ref, tes_ref, tyes_ref, mrows_ref,
                  we_ref, be_ref, out_ref):
    hl = hls_ref[:, :NE]
    hmax = jnp.max(hl, axis=1, keepdims=True)
    he = jnp.exp(hl - hmax)
    hp = he / jnp.sum(he, axis=1, keepdims=True)

    acc = jnp.dot(seqs_ref[...].astype(jnp.bfloat16),
                  we_ref[:D].astype(jnp.bfloat16),
                  preferred_element_type=jnp.float32)
    acc = acc + jnp.dot(sss_ref[...].astype(jnp.bfloat16),
                        we_ref[D:2 * D].astype(jnp.bfloat16),
                        preferred_element_type=jnp.float32)
    acc = acc + jnp.dot(hp.astype(jnp.bfloat16),
                        we_ref[2 * D:].astype(jnp.bfloat16),
                        preferred_element_type=jnp.float32)
    acc = acc + be_ref[...] + tyes_ref[...] + tes_ref[...]
    out_ref[...] = acc * mrows_ref[:, :1]


def _entries(seq_sel, ss_sel, hl_sel, te_sel, tye_sel, mrow_sel, We, be):
    return pl.pallas_call(
        _entries_body,
        out_shape=jax.ShapeDtypeStruct((NSEL, D), jnp.float32),
    )(seq_sel, ss_sel, hl_sel, te_sel, tye_sel, mrow_sel, We,
      be.reshape(1, D))


@functools.partial(jax.jit, static_argnums=())
def kernel(sequence, state_summary, holder_logits, z_per_step, W1, b1, W2, b2,
           Wp, bp, Wn, bn, type_embed, time_embed, We, be):
    seq2d = sequence.reshape(NROWS, D)
    etl2d, prev2d, next2d, nn3, zb3, ty3, hlp = _heads(
        seq2d, z_per_step.reshape(NROWS, NE), holder_logits.reshape(NROWS, NE),
        W1, b1.reshape(1, D), W2, b2.reshape(1, NT),
        Wp, bp.reshape(1, NE), Wn, bn.reshape(1, NE))

    (scores, mask_i, times, type_ids, idx_flat, idx_time, idx_ty,
     idx_msk) = _select(nn3, zb3, ty3)

    mrow_tbl = jnp.concatenate(
        [jnp.zeros((1, 128), jnp.float32), jnp.ones((1, 128), jnp.float32)])
    seq_sel, ss_sel, hl_sel, te_sel, tye_sel, mrow_sel = _sc_gather(
        seq2d, state_summary.reshape(NROWS, D), hlp, time_embed, type_embed,
        mrow_tbl, idx_flat, idx_time, idx_ty, idx_msk)

    ent2d = _entries(seq_sel, ss_sel, hl_sel, te_sel, tye_sel, mrow_sel,
                     We, be)

    return (ent2d.reshape(B, ME, D), mask_i.astype(jnp.bool_), times,
            type_ids, etl2d.reshape(B, T, NT), prev2d.reshape(B, T, NE),
            next2d.reshape(B, T, NE), scores)
